# Initial kernel scaffold; baseline (speedup 1.0000x reference)
import jax
import jax.numpy as jnp
from jax.experimental import pallas as pl


def kernel(node_type, velocity, mesh_pos, srcs, dsts, params):
    raise NotImplementedError("write your pallas kernel here")


# speedup vs baseline: 2.2444x; 2.2444x over previous
import functools

import jax
import jax.numpy as jnp
from jax import lax
from jax.experimental import pallas as pl
from jax.experimental.pallas import tpu as pltpu
from jax.experimental.pallas import tpu_sc as plsc

NN = 50000
EE = 800000
LAT = 64
STEPS = 15

NC, NS = 2, 16
NW = NC * NS
CH = 512
SUB = 128
EP = NW * 49 * CH
RPW = EP // NW
NCH = RPW // CH

NPAD = 50048
TRASH = NPAD
ACC_R = NPAD + 8
RPS = NPAD // NS
HALF = LAT // NC

BE = 5000
BN = 5000

_f32 = jnp.float32


def _sc_mesh():
    return plsc.VectorSubcoreMesh(core_axis_name="c", subcore_axis_name="s")


def _gather2_body(ta, tb, sidx_h, didx_h, oa, ob, sidxv, didxv, bufa, bufb, sem):
    wid = lax.axis_index("s") * NC + lax.axis_index("c")

    def chunk(j, carry):
        base = pl.multiple_of(wid * RPW + j * CH, CH)
        pltpu.sync_copy(sidx_h.at[pl.ds(base, CH)], sidxv)
        pltpu.sync_copy(didx_h.at[pl.ds(base, CH)], didxv)
        ds_ = []
        for k in range(CH // SUB):
            sl = pl.ds(k * SUB, SUB)
            ds_.append(pltpu.async_copy(ta.at[sidxv.at[sl]], bufa.at[sl], sem))
            ds_.append(pltpu.async_copy(tb.at[didxv.at[sl]], bufb.at[sl], sem))
        for d in ds_:
            d.wait()
        pltpu.sync_copy(bufa, oa.at[pl.ds(base, CH), :])
        pltpu.sync_copy(bufb, ob.at[pl.ds(base, CH), :])
        return carry

    lax.fori_loop(0, NCH, chunk, 0)


def _gather2(ta, tb, sidx, didx):
    w = ta.shape[1]
    call = pl.kernel(
        _gather2_body,
        out_type=(jax.ShapeDtypeStruct((EP, w), _f32),
                  jax.ShapeDtypeStruct((EP, w), _f32)),
        mesh=_sc_mesh(),
        scratch_types=[
            pltpu.VMEM((CH,), jnp.int32),
            pltpu.VMEM((CH,), jnp.int32),
            pltpu.VMEM((CH, w), _f32),
            pltpu.VMEM((CH, w), _f32),
            pltpu.SemaphoreType.DMA,
        ],
        compiler_params=pltpu.CompilerParams(use_tc_tiling_on_sc=False),
    )
    return call(ta, tb, sidx, didx)


def _scatter_body(e_h, didx_h, z_h, agg_h, didxv, ebuf, acc):
    c = lax.axis_index("c")
    s = lax.axis_index("s")

    @pl.when(s == 0)
    def _zero():
        pltpu.sync_copy(z_h, acc)

    plsc.subcore_barrier()

    def chunk(j, carry):
        base = pl.multiple_of(s * (EP // NS) + j * CH, CH)
        for k in range(CH // SUB):
            pltpu.sync_copy(didx_h.at[pl.ds(base + k * SUB, SUB)], didxv.at[k])
        pltpu.sync_copy(e_h.at[pl.ds(base, CH), pl.ds(c * HALF, HALF)], ebuf)
        for k in range(CH // SUB):
            pltpu.sync_copy(ebuf.at[pl.ds(k * SUB, SUB), :],
                            acc.at[didxv.at[k]], add=True)
        return carry

    lax.fori_loop(0, EP // NS // CH, chunk, 0)
    plsc.subcore_barrier()
    r0 = pl.multiple_of(s * RPS, 8)
    pltpu.sync_copy(acc.at[pl.ds(r0, RPS), :],
                    agg_h.at[pl.ds(r0, RPS), pl.ds(c * HALF, HALF)])


def _seg_scatter(e, didx, zacc):
    call = pl.kernel(
        _scatter_body,
        out_type=jax.ShapeDtypeStruct((NPAD, LAT), _f32),
        mesh=_sc_mesh(),
        scratch_types=[
            pltpu.VMEM((CH // SUB, SUB), jnp.int32),
            pltpu.VMEM((CH, HALF), _f32),
            pltpu.VMEM_SHARED((ACC_R, HALF), _f32),
        ],
        compiler_params=pltpu.CompilerParams(use_tc_tiling_on_sc=False),
    )
    return call(e, didx, zacc)


def _ln(y, g, b):
    mu = jnp.mean(y, axis=-1, keepdims=True)
    yc = y - mu
    var = jnp.mean(yc * yc, axis=-1, keepdims=True)
    return yc * lax.rsqrt(var + 1e-5) * g + b


def _wspec(shape):
    return pl.BlockSpec(shape, lambda i: (0,) * len(shape))


def _bspec(rows, cols):
    return pl.BlockSpec((rows, cols), lambda i: (i, 0))


def _node0_body(nt_ref, vel_ref, w1v, w1o, b1, w2, b2, g, b, w1s, w1d,
                v_ref, p_ref, q_ref):
    nt = nt_ref[:]
    oh = (nt == lax.broadcasted_iota(jnp.int32, (BN, 9), 1)).astype(_f32)
    x = vel_ref[:] @ w1v[:] + oh @ w1o[:] + b1[:]
    h = jnp.maximum(x, 0.0)
    y = h @ w2[:] + b2[:]
    v = _ln(y, g[:], b[:])
    v_ref[:] = v
    p_ref[:] = v @ w1s[:]
    q_ref[:] = v @ w1d[:]


def _node0(nt, vel, w1v, w1o, b1, w2, b2, g, b, w1s, w1d):
    return pl.pallas_call(
        _node0_body,
        grid=(NN // BN,),
        in_specs=[_bspec(BN, 1), _bspec(BN, 2),
                  _wspec((2, LAT)), _wspec((9, LAT)), _wspec((1, LAT)),
                  _wspec((LAT, LAT)), _wspec((1, LAT)), _wspec((1, LAT)),
                  _wspec((1, LAT)), _wspec((LAT, LAT)), _wspec((LAT, LAT))],
        out_specs=(_bspec(BN, LAT),) * 3,
        out_shape=(jax.ShapeDtypeStruct((NN, LAT), _f32),) * 3,
    )(nt, vel, w1v, w1o, b1, w2, b2, g, b, w1s, w1d)


def _enc_body(mps_ref, mpd_ref, w1xy, w1n, b1, w2, b2, g, b, e_ref):
    rel = mps_ref[:] - mpd_ref[:]
    r = jnp.sqrt(jnp.sum(rel * rel, axis=-1, keepdims=True))
    x = rel @ w1xy[:] + r * w1n[:] + b1[:]
    h = jnp.maximum(x, 0.0)
    y = h @ w2[:] + b2[:]
    e_ref[:] = _ln(y, g[:], b[:])


def _edge_enc(mps, mpd, w1xy, w1n, b1, w2, b2, g, b):
    return pl.pallas_call(
        _enc_body,
        grid=(EE // BE,),
        in_specs=[_bspec(BE, 2), _bspec(BE, 2),
                  _wspec((2, LAT)), _wspec((1, LAT)), _wspec((1, LAT)),
                  _wspec((LAT, LAT)), _wspec((1, LAT)), _wspec((1, LAT)),
                  _wspec((1, LAT))],
        out_specs=_bspec(BE, LAT),
        out_shape=jax.ShapeDtypeStruct((EP, LAT), _f32),
    )(mps, mpd, w1xy, w1n, b1, w2, b2, g, b)


def _edgestep_body(e_ref, hp_ref, hq_ref, w1e, b1, w2, b2, g, b, eo_ref):
    e = e_ref[:]
    x = e @ w1e[:] + hp_ref[:] + hq_ref[:] + b1[:]
    h = jnp.maximum(x, 0.0)
    y = h @ w2[:] + b2[:]
    eo_ref[:] = e + _ln(y, g[:], b[:])


def _edge_step(e, hp, hq, w1e, b1, w2, b2, g, b):
    return pl.pallas_call(
        _edgestep_body,
        grid=(EE // BE,),
        in_specs=[_bspec(BE, LAT)] * 3 +
                 [_wspec((LAT, LAT)), _wspec((1, LAT)), _wspec((LAT, LAT)),
                  _wspec((1, LAT)), _wspec((1, LAT)), _wspec((1, LAT))],
        out_specs=_bspec(BE, LAT),
        out_shape=jax.ShapeDtypeStruct((EP, LAT), _f32),
    )(e, hp, hq, w1e, b1, w2, b2, g, b)


def _nodestep_body(v_ref, agg_ref, w1v, w1a, b1, w2, b2, g, b, w1s, w1d,
                   vo_ref, po_ref, qo_ref):
    v = v_ref[:]
    x = v @ w1v[:] + agg_ref[:] @ w1a[:] + b1[:]
    h = jnp.maximum(x, 0.0)
    y = h @ w2[:] + b2[:]
    vn = v + _ln(y, g[:], b[:])
    vo_ref[:] = vn
    po_ref[:] = vn @ w1s[:]
    qo_ref[:] = vn @ w1d[:]


def _node_step(v, agg, w1v, w1a, b1, w2, b2, g, b, w1s, w1d):
    return pl.pallas_call(
        _nodestep_body,
        grid=(NN // BN,),
        in_specs=[_bspec(BN, LAT), _bspec(BN, LAT),
                  _wspec((LAT, LAT)), _wspec((LAT, LAT)), _wspec((1, LAT)),
                  _wspec((LAT, LAT)), _wspec((1, LAT)), _wspec((1, LAT)),
                  _wspec((1, LAT)), _wspec((LAT, LAT)), _wspec((LAT, LAT))],
        out_specs=(_bspec(BN, LAT),) * 3,
        out_shape=(jax.ShapeDtypeStruct((NN, LAT), _f32),) * 3,
    )(v, agg, w1v, w1a, b1, w2, b2, g, b, w1s, w1d)


def _nodelast_body(v_ref, agg_ref, w1v, w1a, b1, w2, b2, g, b, dw1, db1,
                   dw2, db2, out_ref):
    v = v_ref[:]
    x = v @ w1v[:] + agg_ref[:] @ w1a[:] + b1[:]
    h = jnp.maximum(x, 0.0)
    y = h @ w2[:] + b2[:]
    vn = v + _ln(y, g[:], b[:])
    hd = jnp.maximum(vn @ dw1[:] + db1[:], 0.0)
    out_ref[:] = hd @ dw2[:] + db2[:]


def _node_last(v, agg, w1v, w1a, b1, w2, b2, g, b, dw1, db1, dw2, db2):
    return pl.pallas_call(
        _nodelast_body,
        grid=(NN // BN,),
        in_specs=[_bspec(BN, LAT), _bspec(BN, LAT),
                  _wspec((LAT, LAT)), _wspec((LAT, LAT)), _wspec((1, LAT)),
                  _wspec((LAT, LAT)), _wspec((1, LAT)), _wspec((1, LAT)),
                  _wspec((1, LAT)), _wspec((LAT, LAT)), _wspec((1, LAT)),
                  _wspec((LAT, 2)), _wspec((1, 2))],
        out_specs=_bspec(BN, 2),
        out_shape=jax.ShapeDtypeStruct((NN, 2), _f32),
    )(v, agg, w1v, w1a, b1, w2, b2, g, b, dw1, db1, dw2, db2)


def _row(x):
    return x.reshape(1, -1).astype(_f32)


def kernel(node_type, velocity, mesh_pos, srcs, dsts, params):
    p = params
    srcs_p = jnp.concatenate(
        [srcs.astype(jnp.int32), jnp.zeros((EP - EE,), jnp.int32)])
    dsts32 = dsts.astype(jnp.int32)
    dsts_p = jnp.concatenate([dsts32, jnp.zeros((EP - EE,), jnp.int32)])
    dsts_s = jnp.concatenate([dsts32, jnp.full((EP - EE,), TRASH, jnp.int32)])
    zacc = jnp.zeros((ACC_R, HALF), _f32)

    nm, nstd = p['node_norm']['mean'], p['node_norm']['std']
    ne = p['node_enc']
    nW1 = ne['W1'] / nstd[:, None]
    nb1 = ne['b1'] - (nm / nstd) @ ne['W1']
    em, estd = p['edge_norm']['mean'], p['edge_norm']['std']
    eenc = p['edge_enc']
    eW1 = eenc['W1'] / estd[:, None]
    eb1 = eenc['b1'] - (em / estd) @ eenc['W1']
    om, ostd = p['out_norm']['mean'], p['out_norm']['std']
    dec = p['dec']
    dW2 = dec['W2'] * ostd[None, :]
    db2 = dec['b2'] * ostd + om

    mp = p['mp']
    ew1e = [st['edge']['W1'][:LAT] for st in mp]
    ew1s = [st['edge']['W1'][LAT:2 * LAT] for st in mp]
    ew1d = [st['edge']['W1'][2 * LAT:] for st in mp]
    nw1v = [st['node']['W1'][:LAT] for st in mp]
    nw1a = [st['node']['W1'][LAT:] for st in mp]

    v, pt, qt = _node0(
        node_type.reshape(NN, 1).astype(jnp.int32), velocity,
        nW1[:2], nW1[2:], _row(nb1), ne['W2'], _row(ne['b2']),
        _row(ne['g']), _row(ne['b']), ew1s[0], ew1d[0])

    mps, mpd = _gather2(mesh_pos, mesh_pos, srcs_p, dsts_p)
    e = _edge_enc(mps, mpd, eW1[:2], eW1[2:3], _row(eb1), eenc['W2'],
                  _row(eenc['b2']), _row(eenc['g']), _row(eenc['b']))

    for t in range(STEPS):
        st = mp[t]
        hp, hq = _gather2(pt, qt, srcs_p, dsts_p)
        e = _edge_step(e, hp, hq, ew1e[t], _row(st['edge']['b1']),
                       st['edge']['W2'], _row(st['edge']['b2']),
                       _row(st['edge']['g']), _row(st['edge']['b']))
        agg = _seg_scatter(e, dsts_s, zacc)
        nb = st['node']
        if t < STEPS - 1:
            v, pt, qt = _node_step(
                v, agg, nw1v[t], nw1a[t], _row(nb['b1']), nb['W2'],
                _row(nb['b2']), _row(nb['g']), _row(nb['b']),
                ew1s[t + 1], ew1d[t + 1])
        else:
            out = _node_last(
                v, agg, nw1v[t], nw1a[t], _row(nb['b1']), nb['W2'],
                _row(nb['b2']), _row(nb['g']), _row(nb['b']),
                dec['W1'], _row(dec['b1']), dW2, _row(db2))
    return out


# speedup vs baseline: 2.4349x; 1.0849x over previous
import functools

import jax
import jax.numpy as jnp
from jax import lax
from jax.experimental import pallas as pl
from jax.experimental.pallas import tpu as pltpu
from jax.experimental.pallas import tpu_sc as plsc

NN = 50000
EE = 800000
LAT = 64
STEPS = 15

NC, NS = 2, 16
NW = NC * NS
CH = 512
SUB = 128
EP = NW * 49 * CH
RPW = EP // NW
NCH = RPW // CH

NPAD = 50048
TRASH = NPAD
ACC_R = NPAD + 8
RPS = NPAD // NS
HALF = LAT // NC

BE = 5000
BN = 5000

_f32 = jnp.float32


def _sc_mesh():
    return plsc.VectorSubcoreMesh(core_axis_name="c", subcore_axis_name="s")


def _gather2_body(ta, tb, sidx_h, didx_h, oa, ob, sidxv, didxv, bufa, bufb, sem):
    wid = lax.axis_index("s") * NC + lax.axis_index("c")

    def chunk(j, carry):
        base = pl.multiple_of(wid * RPW + j * CH, CH)
        pltpu.sync_copy(sidx_h.at[pl.ds(base, CH)], sidxv)
        pltpu.sync_copy(didx_h.at[pl.ds(base, CH)], didxv)
        ds_ = []
        for k in range(CH // SUB):
            sl = pl.ds(k * SUB, SUB)
            ds_.append(pltpu.async_copy(ta.at[sidxv.at[sl]], bufa.at[sl], sem))
            ds_.append(pltpu.async_copy(tb.at[didxv.at[sl]], bufb.at[sl], sem))
        for d in ds_:
            d.wait()
        pltpu.sync_copy(bufa, oa.at[pl.ds(base, CH), :])
        pltpu.sync_copy(bufb, ob.at[pl.ds(base, CH), :])
        return carry

    lax.fori_loop(0, NCH, chunk, 0)


def _gather2(ta, tb, sidx, didx):
    w = ta.shape[1]
    dt = ta.dtype
    call = pl.kernel(
        _gather2_body,
        out_type=(jax.ShapeDtypeStruct((EP, w), dt),
                  jax.ShapeDtypeStruct((EP, w), dt)),
        mesh=_sc_mesh(),
        scratch_types=[
            pltpu.VMEM((CH,), jnp.int32),
            pltpu.VMEM((CH,), jnp.int32),
            pltpu.VMEM((CH, w), dt),
            pltpu.VMEM((CH, w), dt),
            pltpu.SemaphoreType.DMA,
        ],
        compiler_params=pltpu.CompilerParams(use_tc_tiling_on_sc=False),
    )
    return call(ta, tb, sidx, didx)


def _scatter_body(e_h, didx_h, z_h, agg_h, didxv, ebuf, acc):
    c = lax.axis_index("c")
    s = lax.axis_index("s")

    @pl.when(s == 0)
    def _zero():
        pltpu.sync_copy(z_h, acc)

    plsc.subcore_barrier()

    def chunk(j, carry):
        base = pl.multiple_of(s * (EP // NS) + j * CH, CH)
        for k in range(CH // SUB):
            pltpu.sync_copy(didx_h.at[pl.ds(base + k * SUB, SUB)], didxv.at[k])
        pltpu.sync_copy(e_h.at[pl.ds(base, CH), pl.ds(c * HALF, HALF)], ebuf)
        for k in range(CH // SUB):
            pltpu.sync_copy(ebuf.at[pl.ds(k * SUB, SUB), :],
                            acc.at[didxv.at[k]], add=True)
        return carry

    lax.fori_loop(0, EP // NS // CH, chunk, 0)
    plsc.subcore_barrier()
    r0 = pl.multiple_of(s * RPS, 8)
    pltpu.sync_copy(acc.at[pl.ds(r0, RPS), :],
                    agg_h.at[pl.ds(r0, RPS), pl.ds(c * HALF, HALF)])


def _seg_scatter(e, didx, zacc):
    call = pl.kernel(
        _scatter_body,
        out_type=jax.ShapeDtypeStruct((NPAD, LAT), _f32),
        mesh=_sc_mesh(),
        scratch_types=[
            pltpu.VMEM((CH // SUB, SUB), jnp.int32),
            pltpu.VMEM((CH, HALF), _f32),
            pltpu.VMEM_SHARED((ACC_R, HALF), _f32),
        ],
        compiler_params=pltpu.CompilerParams(use_tc_tiling_on_sc=False),
    )
    return call(e, didx, zacc)


def _rnd_bf16(u):
    return u + jnp.uint32(0x7FFF) + ((u >> 16) & jnp.uint32(1))


def _pack_bf16(x):
    h = x.shape[1] // 2
    u_lo = lax.bitcast_convert_type(x[:, :h], jnp.uint32)
    u_hi = lax.bitcast_convert_type(x[:, h:], jnp.uint32)
    packed = (_rnd_bf16(u_lo) >> 16) | (_rnd_bf16(u_hi) & jnp.uint32(0xFFFF0000))
    return lax.bitcast_convert_type(packed, jnp.int32)


def _unpack_bf16(x32):
    u = lax.bitcast_convert_type(x32, jnp.uint32)
    lo = lax.bitcast_convert_type(u << 16, _f32)
    hi = lax.bitcast_convert_type(u & jnp.uint32(0xFFFF0000), _f32)
    return jnp.concatenate([lo, hi], axis=-1)


def _ln(y, g, b):
    mu = jnp.mean(y, axis=-1, keepdims=True)
    yc = y - mu
    var = jnp.mean(yc * yc, axis=-1, keepdims=True)
    return yc * lax.rsqrt(var + 1e-5) * g + b


def _wspec(shape):
    return pl.BlockSpec(shape, lambda i: (0,) * len(shape))


def _bspec(rows, cols):
    return pl.BlockSpec((rows, cols), lambda i: (i, 0))


def _node0_body(nt_ref, vel_ref, w1v, w1o, b1, w2, b2, g, b, w1s, w1d,
                v_ref, p_ref, q_ref):
    nt = nt_ref[:]
    oh = (nt == lax.broadcasted_iota(jnp.int32, (BN, 9), 1)).astype(_f32)
    x = vel_ref[:] @ w1v[:] + oh @ w1o[:] + b1[:]
    h = jnp.maximum(x, 0.0)
    y = h @ w2[:] + b2[:]
    v = _ln(y, g[:], b[:])
    v_ref[:] = v
    p_ref[:] = _pack_bf16(v @ w1s[:])
    q_ref[:] = _pack_bf16(v @ w1d[:])


def _node0(nt, vel, w1v, w1o, b1, w2, b2, g, b, w1s, w1d):
    return pl.pallas_call(
        _node0_body,
        grid=(NN // BN,),
        in_specs=[_bspec(BN, 1), _bspec(BN, 2),
                  _wspec((2, LAT)), _wspec((9, LAT)), _wspec((1, LAT)),
                  _wspec((LAT, LAT)), _wspec((1, LAT)), _wspec((1, LAT)),
                  _wspec((1, LAT)), _wspec((LAT, LAT)), _wspec((LAT, LAT))],
        out_specs=(_bspec(BN, LAT), _bspec(BN, LAT // 2), _bspec(BN, LAT // 2)),
        out_shape=(jax.ShapeDtypeStruct((NN, LAT), _f32),
                   jax.ShapeDtypeStruct((NN, LAT // 2), jnp.int32),
                   jax.ShapeDtypeStruct((NN, LAT // 2), jnp.int32)),
    )(nt, vel, w1v, w1o, b1, w2, b2, g, b, w1s, w1d)


def _enc_body(mps_ref, mpd_ref, w1xy, w1n, b1, w2, b2, g, b, e_ref):
    xys = lax.bitcast_convert_type(mps_ref[:], _f32)[:, :2]
    xyd = lax.bitcast_convert_type(mpd_ref[:], _f32)[:, :2]
    rel = xys - xyd
    r = jnp.sqrt(jnp.sum(rel * rel, axis=-1, keepdims=True))
    x = rel @ w1xy[:] + r * w1n[:] + b1[:]
    h = jnp.maximum(x, 0.0)
    y = h @ w2[:] + b2[:]
    e_ref[:] = _ln(y, g[:], b[:])


def _edge_enc(mps, mpd, w1xy, w1n, b1, w2, b2, g, b):
    return pl.pallas_call(
        _enc_body,
        grid=(EE // BE,),
        in_specs=[_bspec(BE, LAT // 2), _bspec(BE, LAT // 2),
                  _wspec((2, LAT)), _wspec((1, LAT)), _wspec((1, LAT)),
                  _wspec((LAT, LAT)), _wspec((1, LAT)), _wspec((1, LAT)),
                  _wspec((1, LAT))],
        out_specs=_bspec(BE, LAT),
        out_shape=jax.ShapeDtypeStruct((EP, LAT), _f32),
    )(mps, mpd, w1xy, w1n, b1, w2, b2, g, b)


def _edgestep_body(e_ref, hp_ref, hq_ref, w1e, b1, w2, b2, g, b, eo_ref):
    e = e_ref[:]
    hpq = _unpack_bf16(hp_ref[:]) + _unpack_bf16(hq_ref[:])
    x = e @ w1e[:] + hpq + b1[:]
    h = jnp.maximum(x, 0.0)
    y = h @ w2[:] + b2[:]
    eo_ref[:] = e + _ln(y, g[:], b[:])


def _edge_step(e, hp, hq, w1e, b1, w2, b2, g, b):
    return pl.pallas_call(
        _edgestep_body,
        grid=(EE // BE,),
        in_specs=[_bspec(BE, LAT), _bspec(BE, LAT // 2), _bspec(BE, LAT // 2)] +
                 [_wspec((LAT, LAT)), _wspec((1, LAT)), _wspec((LAT, LAT)),
                  _wspec((1, LAT)), _wspec((1, LAT)), _wspec((1, LAT))],
        out_specs=_bspec(BE, LAT),
        out_shape=jax.ShapeDtypeStruct((EP, LAT), _f32),
    )(e, hp, hq, w1e, b1, w2, b2, g, b)


def _nodestep_body(v_ref, agg_ref, w1v, w1a, b1, w2, b2, g, b, w1s, w1d,
                   vo_ref, po_ref, qo_ref):
    v = v_ref[:]
    x = v @ w1v[:] + agg_ref[:] @ w1a[:] + b1[:]
    h = jnp.maximum(x, 0.0)
    y = h @ w2[:] + b2[:]
    vn = v + _ln(y, g[:], b[:])
    vo_ref[:] = vn
    po_ref[:] = _pack_bf16(vn @ w1s[:])
    qo_ref[:] = _pack_bf16(vn @ w1d[:])


def _node_step(v, agg, w1v, w1a, b1, w2, b2, g, b, w1s, w1d):
    return pl.pallas_call(
        _nodestep_body,
        grid=(NN // BN,),
        in_specs=[_bspec(BN, LAT), _bspec(BN, LAT),
                  _wspec((LAT, LAT)), _wspec((LAT, LAT)), _wspec((1, LAT)),
                  _wspec((LAT, LAT)), _wspec((1, LAT)), _wspec((1, LAT)),
                  _wspec((1, LAT)), _wspec((LAT, LAT)), _wspec((LAT, LAT))],
        out_specs=(_bspec(BN, LAT), _bspec(BN, LAT // 2), _bspec(BN, LAT // 2)),
        out_shape=(jax.ShapeDtypeStruct((NN, LAT), _f32),
                   jax.ShapeDtypeStruct((NN, LAT // 2), jnp.int32),
                   jax.ShapeDtypeStruct((NN, LAT // 2), jnp.int32)),
    )(v, agg, w1v, w1a, b1, w2, b2, g, b, w1s, w1d)


def _nodelast_body(v_ref, agg_ref, w1v, w1a, b1, w2, b2, g, b, dw1, db1,
                   dw2, db2, out_ref):
    v = v_ref[:]
    x = v @ w1v[:] + agg_ref[:] @ w1a[:] + b1[:]
    h = jnp.maximum(x, 0.0)
    y = h @ w2[:] + b2[:]
    vn = v + _ln(y, g[:], b[:])
    hd = jnp.maximum(vn @ dw1[:] + db1[:], 0.0)
    out_ref[:] = hd @ dw2[:] + db2[:]


def _node_last(v, agg, w1v, w1a, b1, w2, b2, g, b, dw1, db1, dw2, db2):
    return pl.pallas_call(
        _nodelast_body,
        grid=(NN // BN,),
        in_specs=[_bspec(BN, LAT), _bspec(BN, LAT),
                  _wspec((LAT, LAT)), _wspec((LAT, LAT)), _wspec((1, LAT)),
                  _wspec((LAT, LAT)), _wspec((1, LAT)), _wspec((1, LAT)),
                  _wspec((1, LAT)), _wspec((LAT, LAT)), _wspec((1, LAT)),
                  _wspec((LAT, 2)), _wspec((1, 2))],
        out_specs=_bspec(BN, 2),
        out_shape=jax.ShapeDtypeStruct((NN, 2), _f32),
    )(v, agg, w1v, w1a, b1, w2, b2, g, b, dw1, db1, dw2, db2)


def _row(x):
    return x.reshape(1, -1).astype(_f32)


def kernel(node_type, velocity, mesh_pos, srcs, dsts, params):
    p = params
    srcs_p = jnp.concatenate(
        [srcs.astype(jnp.int32), jnp.zeros((EP - EE,), jnp.int32)])
    dsts32 = dsts.astype(jnp.int32)
    dsts_p = jnp.concatenate([dsts32, jnp.zeros((EP - EE,), jnp.int32)])
    dsts_s = jnp.concatenate([dsts32, jnp.full((EP - EE,), TRASH, jnp.int32)])
    zacc = jnp.zeros((ACC_R, HALF), _f32)

    nm, nstd = p['node_norm']['mean'], p['node_norm']['std']
    ne = p['node_enc']
    nW1 = ne['W1'] / nstd[:, None]
    nb1 = ne['b1'] - (nm / nstd) @ ne['W1']
    em, estd = p['edge_norm']['mean'], p['edge_norm']['std']
    eenc = p['edge_enc']
    eW1 = eenc['W1'] / estd[:, None]
    eb1 = eenc['b1'] - (em / estd) @ eenc['W1']
    om, ostd = p['out_norm']['mean'], p['out_norm']['std']
    dec = p['dec']
    dW2 = dec['W2'] * ostd[None, :]
    db2 = dec['b2'] * ostd + om

    mp = p['mp']
    ew1e = [st['edge']['W1'][:LAT] for st in mp]
    ew1s = [st['edge']['W1'][LAT:2 * LAT] for st in mp]
    ew1d = [st['edge']['W1'][2 * LAT:] for st in mp]
    nw1v = [st['node']['W1'][:LAT] for st in mp]
    nw1a = [st['node']['W1'][LAT:] for st in mp]

    v, pt, qt = _node0(
        node_type.reshape(NN, 1).astype(jnp.int32), velocity,
        nW1[:2], nW1[2:], _row(nb1), ne['W2'], _row(ne['b2']),
        _row(ne['g']), _row(ne['b']), ew1s[0], ew1d[0])

    mp_tab = jnp.concatenate(
        [lax.bitcast_convert_type(mesh_pos.astype(_f32), jnp.int32),
         jnp.zeros((NN, LAT // 2 - 2), jnp.int32)], axis=1)
    mps, mpd = _gather2(mp_tab, mp_tab, srcs_p, dsts_p)
    e = _edge_enc(mps, mpd, eW1[:2], eW1[2:3], _row(eb1), eenc['W2'],
                  _row(eenc['b2']), _row(eenc['g']), _row(eenc['b']))

    for t in range(STEPS):
        st = mp[t]
        hp, hq = _gather2(pt, qt, srcs_p, dsts_p)
        e = _edge_step(e, hp, hq, ew1e[t], _row(st['edge']['b1']),
                       st['edge']['W2'], _row(st['edge']['b2']),
                       _row(st['edge']['g']), _row(st['edge']['b']))
        agg = _seg_scatter(e, dsts_s, zacc)
        nb = st['node']
        if t < STEPS - 1:
            v, pt, qt = _node_step(
                v, agg, nw1v[t], nw1a[t], _row(nb['b1']), nb['W2'],
                _row(nb['b2']), _row(nb['g']), _row(nb['b']),
                ew1s[t + 1], ew1d[t + 1])
        else:
            out = _node_last(
                v, agg, nw1v[t], nw1a[t], _row(nb['b1']), nb['W2'],
                _row(nb['b2']), _row(nb['g']), _row(nb['b']),
                dec['W1'], _row(dec['b1']), dW2, _row(db2))
    return out


# speedup vs baseline: 2.4986x; 1.0262x over previous
import functools

import jax
import jax.numpy as jnp
from jax import lax
from jax.experimental import pallas as pl
from jax.experimental.pallas import tpu as pltpu
from jax.experimental.pallas import tpu_sc as plsc

NN = 50000
EE = 800000
LAT = 64
STEPS = 15

NC, NS = 2, 16
NW = NC * NS
CH = 256
SUB = 128
EP = 802816
RPW = EP // NW
NCH = RPW // CH

NPAD = 50048
TRASH = NPAD
ACC_R = NPAD + 8
RPS = NPAD // NS
HALF = LAT // NC

BE = 5000
BN = 5000

_f32 = jnp.float32


def _sc_mesh():
    return plsc.VectorSubcoreMesh(core_axis_name="c", subcore_axis_name="s")


def _gather2_body(ta, tb, sidx_h, didx_h, oa, ob, sidxv, didxv, bufa, bufb,
                  semi, semg, semw):
    wid = lax.axis_index("s") * NC + lax.axis_index("c")
    base_w = wid * RPW

    def issue_idx(jj, b):
        base = pl.multiple_of(base_w + jj * CH, CH)
        pltpu.async_copy(sidx_h.at[pl.ds(base, CH)], sidxv.at[b], semi)
        pltpu.async_copy(didx_h.at[pl.ds(base, CH)], didxv.at[b], semi)

    def wait_idx(b):
        pltpu.make_async_copy(sidx_h.at[pl.ds(0, CH)], sidxv.at[b], semi).wait()
        pltpu.make_async_copy(didx_h.at[pl.ds(0, CH)], didxv.at[b], semi).wait()

    def run_gathers(b):
        ds_ = []
        for k in range(CH // SUB):
            sl = pl.ds(k * SUB, SUB)
            ds_.append(pltpu.async_copy(ta.at[sidxv.at[b, sl]],
                                        bufa.at[b, sl], semg))
            ds_.append(pltpu.async_copy(tb.at[didxv.at[b, sl]],
                                        bufb.at[b, sl], semg))
        for d in ds_:
            d.wait()

    def issue_writes(jj, b):
        base = pl.multiple_of(base_w + jj * CH, CH)
        pltpu.async_copy(bufa.at[b], oa.at[pl.ds(base, CH), :], semw)
        pltpu.async_copy(bufb.at[b], ob.at[pl.ds(base, CH), :], semw)

    def wait_writes(b):
        pltpu.make_async_copy(bufa.at[b], oa.at[pl.ds(0, CH), :], semw).wait()
        pltpu.make_async_copy(bufb.at[b], ob.at[pl.ds(0, CH), :], semw).wait()

    issue_idx(0, 0)
    issue_idx(1, 1)
    for b in range(2):
        wait_idx(b)
        run_gathers(b)
        issue_idx(b + 2, b)
        issue_writes(b, b)

    @pl.loop(2, NCH, step=2)
    def _(j):
        for b in range(2):
            jj = j + b
            wait_writes(b)
            wait_idx(b)
            run_gathers(b)
            issue_idx(jnp.minimum(jj + 2, NCH - 1), b)
            issue_writes(jj, b)

    for b in range(2):
        wait_writes(b)
        wait_idx(b)


def _gather2(ta, tb, sidx, didx):
    w = ta.shape[1]
    dt = ta.dtype
    call = pl.kernel(
        _gather2_body,
        out_type=(jax.ShapeDtypeStruct((EP, w), dt),
                  jax.ShapeDtypeStruct((EP, w), dt)),
        mesh=_sc_mesh(),
        scratch_types=[
            pltpu.VMEM((2, CH), jnp.int32),
            pltpu.VMEM((2, CH), jnp.int32),
            pltpu.VMEM((2, CH, w), dt),
            pltpu.VMEM((2, CH, w), dt),
            pltpu.SemaphoreType.DMA,
            pltpu.SemaphoreType.DMA,
            pltpu.SemaphoreType.DMA,
        ],
        compiler_params=pltpu.CompilerParams(use_tc_tiling_on_sc=False),
    )
    return call(ta, tb, sidx, didx)


NCH_S = EP // NS // CH


def _scatter_body(e_h, didx_h, z_h, agg_h, didxv, ebuf, acc, semi, sema):
    c = lax.axis_index("c")
    s = lax.axis_index("s")
    base_s = s * (EP // NS)

    @pl.when(s == 0)
    def _zero():
        pltpu.sync_copy(z_h, acc)

    plsc.subcore_barrier()

    def issue_in(jj, b):
        base = pl.multiple_of(base_s + jj * CH, CH)
        for k in range(CH // SUB):
            pltpu.async_copy(didx_h.at[pl.ds(base + k * SUB, SUB)],
                             didxv.at[2 * b + k], semi)
        pltpu.async_copy(e_h.at[pl.ds(base, CH), pl.ds(c * HALF, HALF)],
                         ebuf.at[b], semi)

    def wait_in(b):
        for k in range(CH // SUB):
            pltpu.make_async_copy(didx_h.at[pl.ds(0, SUB)],
                                  didxv.at[2 * b + k], semi).wait()
        pltpu.make_async_copy(e_h.at[pl.ds(0, CH), pl.ds(0, HALF)],
                              ebuf.at[b], semi).wait()

    def issue_adds(b):
        for k in range(CH // SUB):
            pltpu.async_copy(ebuf.at[b, pl.ds(k * SUB, SUB), :],
                             acc.at[didxv.at[2 * b + k]], sema, add=True)

    def wait_adds(b):
        for k in range(CH // SUB):
            pltpu.make_async_copy(e_h.at[pl.ds(0, SUB), pl.ds(0, HALF)],
                                  ebuf.at[b, pl.ds(k * SUB, SUB), :],
                                  sema).wait()

    issue_in(0, 0)
    issue_in(1, 1)
    wait_in(0)
    issue_adds(0)
    wait_in(1)
    issue_adds(1)
    wait_adds(0)
    issue_in(2, 0)

    @pl.loop(2, NCH_S, step=2)
    def _(j):
        for b in range(2):
            jj = j + b
            wait_in(b)
            issue_adds(b)
            wait_adds(1 - b)
            issue_in(jnp.minimum(jj + 1, NCH_S - 1), 1 - b)

    wait_adds(1)
    wait_in(0)
    plsc.subcore_barrier()
    r0 = pl.multiple_of(s * RPS, 8)
    pltpu.sync_copy(acc.at[pl.ds(r0, RPS), :],
                    agg_h.at[pl.ds(r0, RPS), pl.ds(c * HALF, HALF)])


def _seg_scatter(e, didx, zacc):
    call = pl.kernel(
        _scatter_body,
        out_type=jax.ShapeDtypeStruct((NPAD, LAT), _f32),
        mesh=_sc_mesh(),
        scratch_types=[
            pltpu.VMEM((2 * (CH // SUB), SUB), jnp.int32),
            pltpu.VMEM((2, CH, HALF), _f32),
            pltpu.VMEM_SHARED((ACC_R, HALF), _f32),
            pltpu.SemaphoreType.DMA,
            pltpu.SemaphoreType.DMA,
        ],
        compiler_params=pltpu.CompilerParams(use_tc_tiling_on_sc=False),
    )
    return call(e, didx, zacc)


def _ln(y, g, b):
    mu = jnp.mean(y, axis=-1, keepdims=True)
    yc = y - mu
    var = jnp.mean(yc * yc, axis=-1, keepdims=True)
    return yc * lax.rsqrt(var + 1e-5) * g + b


def _wspec(shape):
    return pl.BlockSpec(shape, lambda i: (0,) * len(shape))


def _bspec(rows, cols):
    return pl.BlockSpec((rows, cols), lambda i: (i, 0))


def _node0_body(nt_ref, vel_ref, w1v, w1o, b1, w2, b2, g, b, w1s, w1d,
                v_ref, p_ref, q_ref):
    nt = nt_ref[:]
    oh = (nt == lax.broadcasted_iota(jnp.int32, (BN, 9), 1)).astype(_f32)
    x = vel_ref[:] @ w1v[:] + oh @ w1o[:] + b1[:]
    h = jnp.maximum(x, 0.0)
    y = h @ w2[:] + b2[:]
    v = _ln(y, g[:], b[:])
    v_ref[:] = v
    p_ref[:] = v @ w1s[:]
    q_ref[:] = v @ w1d[:]


def _node0(nt, vel, w1v, w1o, b1, w2, b2, g, b, w1s, w1d):
    return pl.pallas_call(
        _node0_body,
        grid=(NN // BN,),
        in_specs=[_bspec(BN, 1), _bspec(BN, 2),
                  _wspec((2, LAT)), _wspec((9, LAT)), _wspec((1, LAT)),
                  _wspec((LAT, LAT)), _wspec((1, LAT)), _wspec((1, LAT)),
                  _wspec((1, LAT)), _wspec((LAT, LAT)), _wspec((LAT, LAT))],
        out_specs=(_bspec(BN, LAT),) * 3,
        out_shape=(jax.ShapeDtypeStruct((NN, LAT), _f32),) * 3,
    )(nt, vel, w1v, w1o, b1, w2, b2, g, b, w1s, w1d)


def _enc_body(mps_ref, mpd_ref, w1xy, w1n, b1, w2, b2, g, b, e_ref):
    xys = lax.bitcast_convert_type(mps_ref[:], _f32)[:, :2]
    xyd = lax.bitcast_convert_type(mpd_ref[:], _f32)[:, :2]
    rel = xys - xyd
    r = jnp.sqrt(jnp.sum(rel * rel, axis=-1, keepdims=True))
    x = rel @ w1xy[:] + r * w1n[:] + b1[:]
    h = jnp.maximum(x, 0.0)
    y = h @ w2[:] + b2[:]
    e_ref[:] = _ln(y, g[:], b[:])


def _edge_enc(mps, mpd, w1xy, w1n, b1, w2, b2, g, b):
    return pl.pallas_call(
        _enc_body,
        grid=(EE // BE,),
        in_specs=[_bspec(BE, LAT // 2), _bspec(BE, LAT // 2),
                  _wspec((2, LAT)), _wspec((1, LAT)), _wspec((1, LAT)),
                  _wspec((LAT, LAT)), _wspec((1, LAT)), _wspec((1, LAT)),
                  _wspec((1, LAT))],
        out_specs=_bspec(BE, LAT),
        out_shape=jax.ShapeDtypeStruct((EP, LAT), _f32),
    )(mps, mpd, w1xy, w1n, b1, w2, b2, g, b)


def _edgestep_body(e_ref, hp_ref, hq_ref, w1e, b1, w2, b2, g, b, eo_ref):
    e = e_ref[:]
    x = e @ w1e[:] + hp_ref[:] + hq_ref[:] + b1[:]
    h = jnp.maximum(x, 0.0)
    y = h @ w2[:] + b2[:]
    eo_ref[:] = e + _ln(y, g[:], b[:])


def _edge_step(e, hp, hq, w1e, b1, w2, b2, g, b):
    return pl.pallas_call(
        _edgestep_body,
        grid=(EE // BE,),
        in_specs=[_bspec(BE, LAT)] * 3 +
                 [_wspec((LAT, LAT)), _wspec((1, LAT)), _wspec((LAT, LAT)),
                  _wspec((1, LAT)), _wspec((1, LAT)), _wspec((1, LAT))],
        out_specs=_bspec(BE, LAT),
        out_shape=jax.ShapeDtypeStruct((EP, LAT), _f32),
    )(e, hp, hq, w1e, b1, w2, b2, g, b)


def _nodestep_body(v_ref, agg_ref, w1v, w1a, b1, w2, b2, g, b, w1s, w1d,
                   vo_ref, po_ref, qo_ref):
    v = v_ref[:]
    x = v @ w1v[:] + agg_ref[:] @ w1a[:] + b1[:]
    h = jnp.maximum(x, 0.0)
    y = h @ w2[:] + b2[:]
    vn = v + _ln(y, g[:], b[:])
    vo_ref[:] = vn
    po_ref[:] = vn @ w1s[:]
    qo_ref[:] = vn @ w1d[:]


def _node_step(v, agg, w1v, w1a, b1, w2, b2, g, b, w1s, w1d):
    return pl.pallas_call(
        _nodestep_body,
        grid=(NN // BN,),
        in_specs=[_bspec(BN, LAT), _bspec(BN, LAT),
                  _wspec((LAT, LAT)), _wspec((LAT, LAT)), _wspec((1, LAT)),
                  _wspec((LAT, LAT)), _wspec((1, LAT)), _wspec((1, LAT)),
                  _wspec((1, LAT)), _wspec((LAT, LAT)), _wspec((LAT, LAT))],
        out_specs=(_bspec(BN, LAT),) * 3,
        out_shape=(jax.ShapeDtypeStruct((NN, LAT), _f32),) * 3,
    )(v, agg, w1v, w1a, b1, w2, b2, g, b, w1s, w1d)


def _nodelast_body(v_ref, agg_ref, w1v, w1a, b1, w2, b2, g, b, dw1, db1,
                   dw2, db2, out_ref):
    v = v_ref[:]
    x = v @ w1v[:] + agg_ref[:] @ w1a[:] + b1[:]
    h = jnp.maximum(x, 0.0)
    y = h @ w2[:] + b2[:]
    vn = v + _ln(y, g[:], b[:])
    hd = jnp.maximum(vn @ dw1[:] + db1[:], 0.0)
    out_ref[:] = hd @ dw2[:] + db2[:]


def _node_last(v, agg, w1v, w1a, b1, w2, b2, g, b, dw1, db1, dw2, db2):
    return pl.pallas_call(
        _nodelast_body,
        grid=(NN // BN,),
        in_specs=[_bspec(BN, LAT), _bspec(BN, LAT),
                  _wspec((LAT, LAT)), _wspec((LAT, LAT)), _wspec((1, LAT)),
                  _wspec((LAT, LAT)), _wspec((1, LAT)), _wspec((1, LAT)),
                  _wspec((1, LAT)), _wspec((LAT, LAT)), _wspec((1, LAT)),
                  _wspec((LAT, 2)), _wspec((1, 2))],
        out_specs=_bspec(BN, 2),
        out_shape=jax.ShapeDtypeStruct((NN, 2), _f32),
    )(v, agg, w1v, w1a, b1, w2, b2, g, b, dw1, db1, dw2, db2)


def _row(x):
    return x.reshape(1, -1).astype(_f32)


def kernel(node_type, velocity, mesh_pos, srcs, dsts, params):
    p = params
    srcs_p = jnp.concatenate(
        [srcs.astype(jnp.int32), jnp.zeros((EP - EE,), jnp.int32)])
    dsts32 = dsts.astype(jnp.int32)
    dsts_p = jnp.concatenate([dsts32, jnp.zeros((EP - EE,), jnp.int32)])
    dsts_s = jnp.concatenate([dsts32, jnp.full((EP - EE,), TRASH, jnp.int32)])
    zacc = jnp.zeros((ACC_R, HALF), _f32)

    nm, nstd = p['node_norm']['mean'], p['node_norm']['std']
    ne = p['node_enc']
    nW1 = ne['W1'] / nstd[:, None]
    nb1 = ne['b1'] - (nm / nstd) @ ne['W1']
    em, estd = p['edge_norm']['mean'], p['edge_norm']['std']
    eenc = p['edge_enc']
    eW1 = eenc['W1'] / estd[:, None]
    eb1 = eenc['b1'] - (em / estd) @ eenc['W1']
    om, ostd = p['out_norm']['mean'], p['out_norm']['std']
    dec = p['dec']
    dW2 = dec['W2'] * ostd[None, :]
    db2 = dec['b2'] * ostd + om

    mp = p['mp']
    ew1e = [st['edge']['W1'][:LAT] for st in mp]
    ew1s = [st['edge']['W1'][LAT:2 * LAT] for st in mp]
    ew1d = [st['edge']['W1'][2 * LAT:] for st in mp]
    nw1v = [st['node']['W1'][:LAT] for st in mp]
    nw1a = [st['node']['W1'][LAT:] for st in mp]

    v, pt, qt = _node0(
        node_type.reshape(NN, 1).astype(jnp.int32), velocity,
        nW1[:2], nW1[2:], _row(nb1), ne['W2'], _row(ne['b2']),
        _row(ne['g']), _row(ne['b']), ew1s[0], ew1d[0])

    mp_tab = jnp.concatenate(
        [lax.bitcast_convert_type(mesh_pos.astype(_f32), jnp.int32),
         jnp.zeros((NN, LAT // 2 - 2), jnp.int32)], axis=1)
    mps, mpd = _gather2(mp_tab, mp_tab, srcs_p, dsts_p)
    e = _edge_enc(mps, mpd, eW1[:2], eW1[2:3], _row(eb1), eenc['W2'],
                  _row(eenc['b2']), _row(eenc['g']), _row(eenc['b']))

    for t in range(STEPS):
        st = mp[t]
        hp, hq = _gather2(pt, qt, srcs_p, dsts_p)
        e = _edge_step(e, hp, hq, ew1e[t], _row(st['edge']['b1']),
                       st['edge']['W2'], _row(st['edge']['b2']),
                       _row(st['edge']['g']), _row(st['edge']['b']))
        agg = _seg_scatter(e, dsts_s, zacc)
        nb = st['node']
        if t < STEPS - 1:
            v, pt, qt = _node_step(
                v, agg, nw1v[t], nw1a[t], _row(nb['b1']), nb['W2'],
                _row(nb['b2']), _row(nb['g']), _row(nb['b']),
                ew1s[t + 1], ew1d[t + 1])
        else:
            out = _node_last(
                v, agg, nw1v[t], nw1a[t], _row(nb['b1']), nb['W2'],
                _row(nb['b2']), _row(nb['g']), _row(nb['b']),
                dec['W1'], _row(dec['b1']), dW2, _row(db2))
    return out
